# baseline (speedup 1.0000x reference)
import functools

import jax
import jax.numpy as jnp
from jax import lax
from jax.experimental import pallas as pl
from jax.experimental.pallas import tpu as pltpu
from jax.experimental.pallas import tpu_sc as plsc

B = 16
D = 768
MAXD = 512
NDOCS = B * MAXD
CHUNK = 256
NCHUNK = 34
SROWS = NCHUNK * CHUNK
HALF = MAXD // 2
INV_T = 50.0


def _mm_body(nd_ref, q_ref, doc_ref, out_ref):
    out_ref[...] = lax.dot_general(
        q_ref[...],
        doc_ref[...],
        (((1,), (1,)), ((), ())),
        preferred_element_type=jnp.float32,
    )


def _doc_map(i, nd):
    total = nd[0]
    for k in range(1, B):
        total = total + nd[k]
    nlive_m1 = jnp.maximum((total + CHUNK - 1) // CHUNK - 1, 0)
    return (jnp.minimum(i, nlive_m1), 0)


def _out_map(i, nd):
    total = nd[0]
    for k in range(1, B):
        total = total + nd[k]
    nlive_m1 = jnp.maximum((total + CHUNK - 1) // CHUNK - 1, 0)
    return (0, jnp.minimum(i, nlive_m1))


def _tc_scores(q, doc_embeds, nd):
    return pl.pallas_call(
        _mm_body,
        grid_spec=pltpu.PrefetchScalarGridSpec(
            num_scalar_prefetch=1,
            grid=(NCHUNK,),
            in_specs=[
                pl.BlockSpec((B, D), lambda i, nd: (0, 0)),
                pl.BlockSpec((CHUNK, D), _doc_map),
            ],
            out_specs=pl.BlockSpec((B, CHUNK), _out_map),
        ),
        out_shape=jax.ShapeDtypeStruct((B, SROWS), jnp.float32),
    )(nd, q, doc_embeds)


def _sc_body(st_hbm, nd_hbm, out_hbm, ndbuf, dbuf, outbuf):
    b = lax.axis_index("s")
    half = lax.axis_index("c")

    pltpu.sync_copy(nd_hbm, ndbuf)
    nd_vec = ndbuf[...]
    iota = lax.iota(jnp.int32, 16)
    offs_vec = plsc.cumsum(nd_vec) - nd_vec
    off_b = jnp.sum(jnp.where(iota == b, offs_vec, 0))

    start = b * SROWS + off_b + half * HALF
    astart = (start // 16) * 16
    m = start - astart
    pltpu.sync_copy(st_hbm.at[pl.ds(astart, HALF + 16)], dbuf)

    for g in range(HALF // 16):
        outbuf[pl.ds(g * 16, 16)] = plsc.load_gather(dbuf, [m + g * 16 + iota])
    pltpu.sync_copy(outbuf, out_hbm.at[pl.ds(b * MAXD + half * HALF, HALF)])


def _sc_extract(st_flat, nd):
    kfn = functools.partial(
        pl.kernel,
        mesh=plsc.VectorSubcoreMesh(core_axis_name="c", subcore_axis_name="s"),
        compiler_params=pltpu.CompilerParams(needs_layout_passes=False),
        out_type=jax.ShapeDtypeStruct((NDOCS,), jnp.float32),
        scratch_types=[
            pltpu.VMEM((B,), jnp.int32),
            pltpu.VMEM((HALF + 16,), jnp.float32),
            pltpu.VMEM((HALF,), jnp.float32),
        ],
    )(_sc_body)
    return kfn(st_flat, nd)


def _tc_body(nd_smem, sim_ref, labels_ref, ndv_ref, out_ref):
    sim = sim_ref[...] * INV_T
    ndcol = ndv_ref[...]
    pos = lax.broadcasted_iota(jnp.int32, (B, MAXD), 1)
    mask = pos < ndcol
    sims = jnp.where(mask, sim, -jnp.inf)
    mx = jnp.max(sims, axis=1, keepdims=True)
    mxs = jnp.where(ndcol > 0, mx, 0.0)
    ex = jnp.where(mask, jnp.exp(sims - mxs), 0.0)
    sexp = jnp.sum(ex, axis=1, keepdims=True)
    logz = jnp.log(sexp)

    labels = labels_ref[...]
    pt = jnp.where(mask, labels, 0.0)
    s = jnp.sum(pt, axis=1, keepdims=True) + 1e-9
    pt = pt / s
    logpt = jnp.log(jnp.where(pt > 0, pt, 1.0))
    logsm = sims - mxs - logz
    terms = jnp.where(mask, pt * logpt - pt * logsm, 0.0)
    out_ref[0, 0] = jnp.sum(terms) * (1.0 / B)


def _tc_loss(sim2d, soft_labels, nd):
    return pl.pallas_call(
        _tc_body,
        in_specs=[
            pl.BlockSpec(memory_space=pltpu.SMEM),
            pl.BlockSpec((B, MAXD), lambda: (0, 0)),
            pl.BlockSpec((B, MAXD), lambda: (0, 0)),
            pl.BlockSpec((B, 1), lambda: (0, 0)),
        ],
        out_specs=pl.BlockSpec(memory_space=pltpu.SMEM),
        out_shape=jax.ShapeDtypeStruct((1, 1), jnp.float32),
    )(nd, sim2d, soft_labels, nd.reshape(B, 1))


def kernel(query_embeds, doc_embeds, soft_labels, num_docs_per_sample):
    nd = num_docs_per_sample.astype(jnp.int32)
    st = _tc_scores(query_embeds, doc_embeds, nd)
    simflat = _sc_extract(st.reshape(-1), nd)
    sim2d = simflat.reshape(B, MAXD)
    out = _tc_loss(sim2d, soft_labels, nd)
    return out[0, 0]


# speedup vs baseline: 1.1909x; 1.1909x over previous
import functools

import jax
import jax.numpy as jnp
from jax import lax
from jax.experimental import pallas as pl
from jax.experimental.pallas import tpu as pltpu
from jax.experimental.pallas import tpu_sc as plsc

B = 16
D = 768
MAXD = 512
NDOCS = B * MAXD
CHUNK = 1024
NCHUNK = 9
SROWS = NCHUNK * CHUNK
HALF = MAXD // 2
INV_T = 50.0


def _mm_body(nd_ref, q_ref, doc_ref, out_ref):
    out_ref[...] = lax.dot_general(
        q_ref[...],
        doc_ref[...],
        (((1,), (1,)), ((), ())),
        preferred_element_type=jnp.float32,
    )


def _doc_map(i, nd):
    total = nd[0]
    for k in range(1, B):
        total = total + nd[k]
    nlive_m1 = jnp.maximum((total + CHUNK - 1) // CHUNK - 1, 0)
    return (jnp.minimum(i, nlive_m1), 0)


def _out_map(i, nd):
    total = nd[0]
    for k in range(1, B):
        total = total + nd[k]
    nlive_m1 = jnp.maximum((total + CHUNK - 1) // CHUNK - 1, 0)
    return (0, jnp.minimum(i, nlive_m1))


def _tc_scores(q, doc_embeds, nd):
    return pl.pallas_call(
        _mm_body,
        grid_spec=pltpu.PrefetchScalarGridSpec(
            num_scalar_prefetch=1,
            grid=(NCHUNK,),
            in_specs=[
                pl.BlockSpec((B, D), lambda i, nd: (0, 0)),
                pl.BlockSpec((CHUNK, D), _doc_map),
            ],
            out_specs=pl.BlockSpec((B, CHUNK), _out_map),
        ),
        out_shape=jax.ShapeDtypeStruct((B, SROWS), jnp.float32),
    )(nd, q, doc_embeds)


def _sc_body(st_hbm, nd_hbm, out_hbm, ndbuf, dbuf, outbuf):
    b = lax.axis_index("s")
    half = lax.axis_index("c")

    pltpu.sync_copy(nd_hbm, ndbuf)
    nd_vec = ndbuf[...]
    iota = lax.iota(jnp.int32, 16)
    offs_vec = plsc.cumsum(nd_vec) - nd_vec
    off_b = jnp.sum(jnp.where(iota == b, offs_vec, 0))

    start = b * SROWS + off_b + half * HALF
    astart = (start // 16) * 16
    m = start - astart
    pltpu.sync_copy(st_hbm.at[pl.ds(astart, HALF + 16)], dbuf)

    for g in range(HALF // 16):
        outbuf[pl.ds(g * 16, 16)] = plsc.load_gather(dbuf, [m + g * 16 + iota])
    pltpu.sync_copy(outbuf, out_hbm.at[pl.ds(b * MAXD + half * HALF, HALF)])


def _sc_extract(st_flat, nd):
    kfn = functools.partial(
        pl.kernel,
        mesh=plsc.VectorSubcoreMesh(core_axis_name="c", subcore_axis_name="s"),
        compiler_params=pltpu.CompilerParams(needs_layout_passes=False),
        out_type=jax.ShapeDtypeStruct((NDOCS,), jnp.float32),
        scratch_types=[
            pltpu.VMEM((B,), jnp.int32),
            pltpu.VMEM((HALF + 16,), jnp.float32),
            pltpu.VMEM((HALF,), jnp.float32),
        ],
    )(_sc_body)
    return kfn(st_flat, nd)


def _tc_body(nd_smem, sim_ref, labels_ref, ndv_ref, out_ref):
    sim = sim_ref[...] * INV_T
    ndcol = ndv_ref[...]
    pos = lax.broadcasted_iota(jnp.int32, (B, MAXD), 1)
    mask = pos < ndcol
    sims = jnp.where(mask, sim, -jnp.inf)
    mx = jnp.max(sims, axis=1, keepdims=True)
    mxs = jnp.where(ndcol > 0, mx, 0.0)
    ex = jnp.where(mask, jnp.exp(sims - mxs), 0.0)
    sexp = jnp.sum(ex, axis=1, keepdims=True)
    logz = jnp.log(sexp)

    labels = labels_ref[...]
    pt = jnp.where(mask, labels, 0.0)
    s = jnp.sum(pt, axis=1, keepdims=True) + 1e-9
    pt = pt / s
    logpt = jnp.log(jnp.where(pt > 0, pt, 1.0))
    logsm = sims - mxs - logz
    terms = jnp.where(mask, pt * logpt - pt * logsm, 0.0)
    out_ref[0, 0] = jnp.sum(terms) * (1.0 / B)


def _tc_loss(sim2d, soft_labels, nd):
    return pl.pallas_call(
        _tc_body,
        in_specs=[
            pl.BlockSpec(memory_space=pltpu.SMEM),
            pl.BlockSpec((B, MAXD), lambda: (0, 0)),
            pl.BlockSpec((B, MAXD), lambda: (0, 0)),
            pl.BlockSpec((B, 1), lambda: (0, 0)),
        ],
        out_specs=pl.BlockSpec(memory_space=pltpu.SMEM),
        out_shape=jax.ShapeDtypeStruct((1, 1), jnp.float32),
    )(nd, sim2d, soft_labels, nd.reshape(B, 1))


def kernel(query_embeds, doc_embeds, soft_labels, num_docs_per_sample):
    nd = num_docs_per_sample.astype(jnp.int32)
    st = _tc_scores(query_embeds, doc_embeds, nd)
    simflat = _sc_extract(st.reshape(-1), nd)
    sim2d = simflat.reshape(B, MAXD)
    out = _tc_loss(sim2d, soft_labels, nd)
    return out[0, 0]


# speedup vs baseline: 1.1990x; 1.0068x over previous
import functools

import jax
import jax.numpy as jnp
from jax import lax
from jax.experimental import pallas as pl
from jax.experimental.pallas import tpu as pltpu
from jax.experimental.pallas import tpu_sc as plsc

B = 16
D = 768
MAXD = 512
NDOCS = B * MAXD
CHUNK = 2048
NCHUNK = 5
SROWS = NCHUNK * CHUNK
HALF = MAXD // 2
INV_T = 50.0


def _mm_body(nd_ref, q_ref, doc_ref, out_ref):
    out_ref[...] = lax.dot_general(
        q_ref[...],
        doc_ref[...],
        (((1,), (1,)), ((), ())),
        preferred_element_type=jnp.float32,
    )


def _doc_map(i, nd):
    total = nd[0]
    for k in range(1, B):
        total = total + nd[k]
    nlive_m1 = jnp.maximum((total + CHUNK - 1) // CHUNK - 1, 0)
    return (jnp.minimum(i, nlive_m1), 0)


def _out_map(i, nd):
    total = nd[0]
    for k in range(1, B):
        total = total + nd[k]
    nlive_m1 = jnp.maximum((total + CHUNK - 1) // CHUNK - 1, 0)
    return (0, jnp.minimum(i, nlive_m1))


def _tc_scores(q, doc_embeds, nd):
    return pl.pallas_call(
        _mm_body,
        grid_spec=pltpu.PrefetchScalarGridSpec(
            num_scalar_prefetch=1,
            grid=(NCHUNK,),
            in_specs=[
                pl.BlockSpec((B, D), lambda i, nd: (0, 0)),
                pl.BlockSpec((CHUNK, D), _doc_map),
            ],
            out_specs=pl.BlockSpec((B, CHUNK), _out_map),
        ),
        out_shape=jax.ShapeDtypeStruct((B, SROWS), jnp.float32),
    )(nd, q, doc_embeds)


def _sc_body(st_hbm, nd_hbm, out_hbm, ndbuf, dbuf, outbuf):
    b = lax.axis_index("s")
    half = lax.axis_index("c")

    pltpu.sync_copy(nd_hbm, ndbuf)
    nd_vec = ndbuf[...]
    iota = lax.iota(jnp.int32, 16)
    offs_vec = plsc.cumsum(nd_vec) - nd_vec
    off_b = jnp.sum(jnp.where(iota == b, offs_vec, 0))

    start = b * SROWS + off_b + half * HALF
    astart = (start // 16) * 16
    m = start - astart
    pltpu.sync_copy(st_hbm.at[pl.ds(astart, HALF + 16)], dbuf)

    for g in range(HALF // 16):
        outbuf[pl.ds(g * 16, 16)] = plsc.load_gather(dbuf, [m + g * 16 + iota])
    pltpu.sync_copy(outbuf, out_hbm.at[pl.ds(b * MAXD + half * HALF, HALF)])


def _sc_extract(st_flat, nd):
    kfn = functools.partial(
        pl.kernel,
        mesh=plsc.VectorSubcoreMesh(core_axis_name="c", subcore_axis_name="s"),
        compiler_params=pltpu.CompilerParams(needs_layout_passes=False),
        out_type=jax.ShapeDtypeStruct((NDOCS,), jnp.float32),
        scratch_types=[
            pltpu.VMEM((B,), jnp.int32),
            pltpu.VMEM((HALF + 16,), jnp.float32),
            pltpu.VMEM((HALF,), jnp.float32),
        ],
    )(_sc_body)
    return kfn(st_flat, nd)


def _tc_body(nd_smem, sim_ref, labels_ref, ndv_ref, out_ref):
    sim = sim_ref[...] * INV_T
    ndcol = ndv_ref[...]
    pos = lax.broadcasted_iota(jnp.int32, (B, MAXD), 1)
    mask = pos < ndcol
    sims = jnp.where(mask, sim, -jnp.inf)
    mx = jnp.max(sims, axis=1, keepdims=True)
    mxs = jnp.where(ndcol > 0, mx, 0.0)
    ex = jnp.where(mask, jnp.exp(sims - mxs), 0.0)
    sexp = jnp.sum(ex, axis=1, keepdims=True)
    logz = jnp.log(sexp)

    labels = labels_ref[...]
    pt = jnp.where(mask, labels, 0.0)
    s = jnp.sum(pt, axis=1, keepdims=True) + 1e-9
    pt = pt / s
    logpt = jnp.log(jnp.where(pt > 0, pt, 1.0))
    logsm = sims - mxs - logz
    terms = jnp.where(mask, pt * logpt - pt * logsm, 0.0)
    out_ref[0, 0] = jnp.sum(terms) * (1.0 / B)


def _tc_loss(sim2d, soft_labels, nd):
    return pl.pallas_call(
        _tc_body,
        in_specs=[
            pl.BlockSpec(memory_space=pltpu.SMEM),
            pl.BlockSpec((B, MAXD), lambda: (0, 0)),
            pl.BlockSpec((B, MAXD), lambda: (0, 0)),
            pl.BlockSpec((B, 1), lambda: (0, 0)),
        ],
        out_specs=pl.BlockSpec(memory_space=pltpu.SMEM),
        out_shape=jax.ShapeDtypeStruct((1, 1), jnp.float32),
    )(nd, sim2d, soft_labels, nd.reshape(B, 1))


def kernel(query_embeds, doc_embeds, soft_labels, num_docs_per_sample):
    nd = num_docs_per_sample.astype(jnp.int32)
    st = _tc_scores(query_embeds, doc_embeds, nd)
    simflat = _sc_extract(st.reshape(-1), nd)
    sim2d = simflat.reshape(B, MAXD)
    out = _tc_loss(sim2d, soft_labels, nd)
    return out[0, 0]
